# baseline (speedup 1.0000x reference)
import functools

import numpy as np
import jax
import jax.numpy as jnp
from jax import lax
from jax.experimental import pallas as pl
from jax.experimental.pallas import tpu as pltpu
from jax.experimental.pallas import tpu_sc as plsc

_TARGET = 0

N_ROWS = 1_000_000
ROW = 8
L = 16
NC, NS = 2, 16
NW = NC * NS
LANE = 128

TILES_PER_W = 244
SC_TILES = TILES_PER_W * NW
SC_LANES = SC_TILES * LANE
TC_REM = N_ROWS - SC_LANES
TC_BLOCK = 1024
NCHUNKS = 4
CHUNK_TILES = TILES_PER_W // NCHUNKS
CHUNK_LANES = CHUNK_TILES * LANE
GROUPS_PER_CHUNK = CHUNK_LANES // L
WORDS_PER_W = TILES_PER_W * LANE

_CLS, _SCORE = 0, ROW - 1
_ROWS_CONST = np.array([_CLS, _SCORE], dtype=np.int32)


def _sc_partials_body(
    xt_hbm, rows_hbm, out_hbm, idx_ref, buf0, buf1, buf2, buf3, acc_ref, *sems
):
    cid = lax.axis_index("c")
    sid = lax.axis_index("s")
    wid = sid * NC + cid

    base = pl.multiple_of(wid * WORDS_PER_W, LANE)
    pltpu.sync_copy(rows_hbm, idx_ref)

    bufs = (buf0, buf1, buf2, buf3)

    def make_group_body(buf):
        def group_body(g, acc):
            cls = buf[0, pl.ds(g * L, L)]
            sc = buf[1, pl.ds(g * L, L)]
            keep = cls.astype(jnp.int32) == _TARGET
            return acc + jnp.where(keep, sc, 0.0)

        return group_body

    cps = [
        pltpu.async_copy(
            xt_hbm.at[idx_ref, pl.ds(base + c * CHUNK_LANES, CHUNK_LANES)],
            bufs[c],
            sems[c],
        )
        for c in range(NCHUNKS)
    ]
    acc = jnp.zeros((L,), jnp.float32)
    for c in range(NCHUNKS):
        cps[c].wait()
        acc = plsc.parallel_loop(0, GROUPS_PER_CHUNK, unroll=4, carry=acc)(
            make_group_body(bufs[c])
        )
    acc_ref[...] = acc
    pltpu.sync_copy(acc_ref, out_hbm.at[wid])


_sc_partials = pl.kernel(
    _sc_partials_body,
    out_type=jax.ShapeDtypeStruct((NW, L), jnp.float32),
    mesh=plsc.VectorSubcoreMesh(
        core_axis_name="c", subcore_axis_name="s", num_cores=NC, num_subcores=NS
    ),
    compiler_params=pltpu.CompilerParams(
        needs_layout_passes=False, use_tc_tiling_on_sc=True
    ),
    scratch_types=[
        pltpu.VMEM((2,), jnp.int32),
        pltpu.VMEM((2, CHUNK_LANES), jnp.float32),
        pltpu.VMEM((2, CHUNK_LANES), jnp.float32),
        pltpu.VMEM((2, CHUNK_LANES), jnp.float32),
        pltpu.VMEM((2, CHUNK_LANES), jnp.float32),
        pltpu.VMEM((L,), jnp.float32),
        pltpu.SemaphoreType.DMA,
        pltpu.SemaphoreType.DMA,
        pltpu.SemaphoreType.DMA,
        pltpu.SemaphoreType.DMA,
    ],
)


def _finish_body(p_ref, x_ref, o_ref):
    cls = x_ref[_CLS : _CLS + 1, :]
    sc = x_ref[_SCORE : _SCORE + 1, :]
    valid = lax.broadcasted_iota(jnp.int32, (1, TC_BLOCK), 1) < TC_REM
    keep = jnp.logical_and(cls.astype(jnp.int32) == _TARGET, valid)
    tail = jnp.sum(jnp.where(keep, sc, 0.0))
    o_ref[0, 0] = -(jnp.sum(p_ref[...]) + tail)


_finish = pl.pallas_call(
    _finish_body,
    out_shape=jax.ShapeDtypeStruct((1, 1), jnp.float32),
    grid=(1,),
    in_specs=[
        pl.BlockSpec((NW, L), lambda i: (0, 0)),
        pl.BlockSpec((ROW, TC_BLOCK), lambda i: (0, SC_LANES // TC_BLOCK)),
    ],
    out_specs=pl.BlockSpec((1, 1), lambda i: (0, 0), memory_space=pltpu.SMEM),
)


@jax.jit
def kernel(box3d_branch):
    xt = box3d_branch.T
    partials = _sc_partials(xt, _ROWS_CONST)
    return _finish(partials, xt)[0, 0]


# speedup vs baseline: 1.0051x; 1.0051x over previous
import functools

import numpy as np
import jax
import jax.numpy as jnp
from jax import lax
from jax.experimental import pallas as pl
from jax.experimental.pallas import tpu as pltpu
from jax.experimental.pallas import tpu_sc as plsc

_TARGET = 0

N_ROWS = 1_000_000
ROW = 8
L = 16
NC, NS = 2, 16
NW = NC * NS
LANE = 128

TILES_PER_W = 244
SC_TILES = TILES_PER_W * NW
SC_LANES = SC_TILES * LANE
TC_REM = N_ROWS - SC_LANES
TC_BLOCK = 1024
NCHUNKS = 4
CHUNK_TILES = TILES_PER_W // NCHUNKS
CHUNK_LANES = CHUNK_TILES * LANE
GROUPS_PER_CHUNK = CHUNK_LANES // L
WORDS_PER_W = TILES_PER_W * LANE

_CLS, _SCORE = 0, ROW - 1
_ROWS_CONST = np.array([_CLS, _SCORE], dtype=np.int32)


def _sc_partials_body(
    xt_hbm, rows_hbm, out_hbm, idx_ref, buf0, buf1, buf2, buf3, acc_ref, *sems
):
    cid = lax.axis_index("c")
    sid = lax.axis_index("s")
    wid = sid * NC + cid

    base = pl.multiple_of(wid * WORDS_PER_W, LANE)
    pltpu.sync_copy(rows_hbm, idx_ref)

    bufs = (buf0, buf1, buf2, buf3)

    def make_group_body(buf):
        def group_body(g, acc):
            cls = buf[0, pl.ds(g * L, L)]
            sc = buf[1, pl.ds(g * L, L)]
            keep = cls.astype(jnp.int32) == _TARGET
            return acc + jnp.where(keep, sc, 0.0)

        return group_body

    cps = [
        pltpu.async_copy(
            xt_hbm.at[idx_ref, pl.ds(base + c * CHUNK_LANES, CHUNK_LANES)],
            bufs[c],
            sems[c],
        )
        for c in range(NCHUNKS)
    ]
    acc = jnp.zeros((L,), jnp.float32)
    for c in range(NCHUNKS):
        cps[c].wait()
        acc = plsc.parallel_loop(0, GROUPS_PER_CHUNK, unroll=4, carry=acc)(
            make_group_body(bufs[c])
        )
    acc_ref[...] = acc
    pltpu.sync_copy(acc_ref, out_hbm.at[wid])


_sc_partials = pl.kernel(
    _sc_partials_body,
    out_type=jax.ShapeDtypeStruct((NW, L), jnp.float32),
    mesh=plsc.VectorSubcoreMesh(
        core_axis_name="c", subcore_axis_name="s", num_cores=NC, num_subcores=NS
    ),
    compiler_params=pltpu.CompilerParams(
        needs_layout_passes=False,
        use_tc_tiling_on_sc=True,
        skip_device_barrier=True,
    ),
    scratch_types=[
        pltpu.VMEM((2,), jnp.int32),
        pltpu.VMEM((2, CHUNK_LANES), jnp.float32),
        pltpu.VMEM((2, CHUNK_LANES), jnp.float32),
        pltpu.VMEM((2, CHUNK_LANES), jnp.float32),
        pltpu.VMEM((2, CHUNK_LANES), jnp.float32),
        pltpu.VMEM((L,), jnp.float32),
        pltpu.SemaphoreType.DMA,
        pltpu.SemaphoreType.DMA,
        pltpu.SemaphoreType.DMA,
        pltpu.SemaphoreType.DMA,
    ],
)


def _finish_body(p_ref, x_ref, o_ref):
    cls = x_ref[_CLS : _CLS + 1, :]
    sc = x_ref[_SCORE : _SCORE + 1, :]
    valid = lax.broadcasted_iota(jnp.int32, (1, TC_BLOCK), 1) < TC_REM
    keep = jnp.logical_and(cls.astype(jnp.int32) == _TARGET, valid)
    tail = jnp.sum(jnp.where(keep, sc, 0.0))
    o_ref[0, 0] = -(jnp.sum(p_ref[...]) + tail)


_finish = pl.pallas_call(
    _finish_body,
    out_shape=jax.ShapeDtypeStruct((1, 1), jnp.float32),
    grid=(1,),
    in_specs=[
        pl.BlockSpec((NW, L), lambda i: (0, 0)),
        pl.BlockSpec((ROW, TC_BLOCK), lambda i: (0, SC_LANES // TC_BLOCK)),
    ],
    out_specs=pl.BlockSpec((1, 1), lambda i: (0, 0), memory_space=pltpu.SMEM),
)


@jax.jit
def kernel(box3d_branch):
    xt = box3d_branch.T
    partials = _sc_partials(xt, _ROWS_CONST)
    return _finish(partials, xt)[0, 0]
